# Initial kernel scaffold; baseline (speedup 1.0000x reference)
import jax
import jax.numpy as jnp
from jax.experimental import pallas as pl


def kernel(coordinates, signal):
    raise NotImplementedError("write your pallas kernel here")


# speedup vs baseline: 4.2261x; 4.2261x over previous
import functools

import jax
import jax.numpy as jnp
from jax.experimental import pallas as pl

_ITERS = 3
_T = 1.0
_DELTA = 1.0
_K = 8
_V = 2048
_N = 4096
_RB = 256


def _gravity3(coords):
    for _ in range(_ITERS):
        centroid = jnp.mean(coords, axis=1)
        dirs = centroid[:, None, :] - coords
        dist = jnp.sqrt(jnp.sum(dirs * dirs, axis=-1) + 1e-12)
        dirs = dirs / dist[..., None]
        coords = coords + ((_T * _T) / 2.0 * (dist - _DELTA))[..., None] * dirs
    return coords


def _dist_body(cb_ref, cbt_ref, sq_ref, sqt_ref, d_ref):
    dot = jnp.dot(cb_ref[0], cbt_ref[0], preferred_element_type=jnp.float32)
    d2 = (sq_ref[0] + sqt_ref[0]) - 2.0 * dot
    d_ref[0] = jnp.sqrt(jnp.maximum(d2, 0.0) + 1e-12)


def _sel_body(d_ref, idx_ref):
    di = jax.lax.bitcast_convert_type(d_ref[0], jnp.int32)
    big = jnp.int32(jnp.iinfo(jnp.int32).max)
    iota = jax.lax.broadcasted_iota(jnp.int32, (_N, _RB), 0)
    for j in range(_K):
        minv = jnp.min(di, axis=0, keepdims=True)
        cand = jnp.where(di == minv, iota, _N)
        idxj = jnp.min(cand, axis=0, keepdims=True)
        idx_ref[0, pl.ds(j, 1), :] = idxj
        di = jnp.where(cand == idxj, big, di)


@functools.partial(jax.jit, static_argnames=())
def _knn_pallas(cb, cbt, sq_c, sq_t):
    b = cb.shape[0]
    nblk = _N // _RB
    dt = pl.pallas_call(
        _dist_body,
        grid=(b, nblk),
        in_specs=[
            pl.BlockSpec((1, _N, 8), lambda b, i: (b, 0, 0)),
            pl.BlockSpec((1, 8, _RB), lambda b, i: (b, 0, i)),
            pl.BlockSpec((1, _N, 1), lambda b, i: (b, 0, 0)),
            pl.BlockSpec((1, 1, _RB), lambda b, i: (b, 0, i)),
        ],
        out_specs=pl.BlockSpec((1, _N, _RB), lambda b, i: (b, 0, i)),
        out_shape=jax.ShapeDtypeStruct((b, _N, _N), jnp.float32),
    )(cb, cbt, sq_c, sq_t)
    return pl.pallas_call(
        _sel_body,
        grid=(b, nblk),
        in_specs=[pl.BlockSpec((1, _N, _RB), lambda b, i: (b, 0, i))],
        out_specs=pl.BlockSpec((1, _K, _RB), lambda b, i: (b, 0, i)),
        out_shape=jax.ShapeDtypeStruct((b, _K, _N), jnp.int32),
    )(dt)


def kernel(coordinates, signal):
    coords = _gravity3(coordinates)
    b = coords.shape[0]
    cb = jnp.pad(coords.astype(jnp.bfloat16), ((0, 0), (0, 0), (0, 5)))
    cbt = jnp.swapaxes(cb, 1, 2)
    sq = jnp.sum(coords * coords, axis=-1)
    sq_c = sq[:, :, None]
    sq_t = sq[:, None, :]
    idx_t = _knn_pallas(cb, cbt, sq_c, sq_t)
    nbh_idx = jnp.swapaxes(idx_t, 1, 2)

    neighborhoods = jax.vmap(lambda cc, i: cc[i])(coords, nbh_idx)
    diff = neighborhoods[:, :, :, None, :] - neighborhoods[:, :, None, :, :]
    dmat = jnp.sqrt(jnp.sum(diff * diff, axis=-1) + 1e-12)
    densities = jnp.sum(dmat, axis=(-1, -2))
    keep = jnp.argsort(densities, axis=-1)[:, :_V]
    idx_kept = jax.vmap(lambda a, kk: a[kk])(nbh_idx, keep)
    sig_gath = jax.vmap(lambda s, i: s[i])(signal, idx_kept)
    new_signal = jnp.mean(sig_gath, axis=-2)
    new_coords = jax.vmap(lambda cc, kk: cc[kk])(coords, keep)
    return new_coords, new_signal
